# baseline (speedup 1.0000x reference)
import math
import functools

import numpy as np
import jax
import jax.numpy as jnp
from jax import lax
from jax.experimental import pallas as pl
from jax.experimental.pallas import tpu as pltpu
from jax.experimental.pallas import tpu_sc as plsc

D_MODEL = 1024
MAX_LEN = 8192
BATCH = 4
LANES = 16
NUM_CORES = 2
NUM_SUBCORES = 16
NUM_WORKERS = NUM_CORES * NUM_SUBCORES
SEQ_PER_WORKER = MAX_LEN // NUM_WORKERS
CHUNK = 32
G = 16
QPC = CHUNK // G
CHUNKS_PER_WORKER = SEQ_PER_WORKER // CHUNK
STEPS = CHUNKS_PER_WORKER * BATCH
ROWS_PER_BATCH = MAX_LEN // CHUNK
NQ = MAX_LEN // G
NBUF = 2


def _make_pe_factors():
    w = np.exp(np.arange(0, D_MODEL, 2, dtype=np.float64)
               * (-math.log(10000.0) / D_MODEL))
    th = (G * np.arange(NQ, dtype=np.float64))[:, None] * w[None, :]
    ph = np.arange(G, dtype=np.float64)[:, None] * w[None, :]
    a = np.zeros((NQ, D_MODEL), np.float32)
    b = np.zeros((NQ, D_MODEL), np.float32)
    c = np.zeros((G, D_MODEL), np.float32)
    s = np.zeros((G, D_MODEL), np.float32)
    a[:, 0::2], a[:, 1::2] = np.sin(th), np.cos(th)
    b[:, 0::2], b[:, 1::2] = np.cos(th), -np.sin(th)
    c[:, 0::2], c[:, 1::2] = np.cos(ph), np.cos(ph)
    s[:, 0::2], s[:, 1::2] = np.sin(ph), np.sin(ph)
    ab = np.stack([a, b], axis=1).reshape(NQ * 2 * D_MODEL)
    cs = np.concatenate([c, s], axis=0).reshape(2 * G * D_MODEL)
    return ab, cs


_mesh = plsc.VectorSubcoreMesh(
    core_axis_name="c", subcore_axis_name="s",
    num_cores=NUM_CORES, num_subcores=NUM_SUBCORES)

AB_CHUNK = QPC * 2 * D_MODEL


@functools.partial(
    pl.kernel,
    out_type=jax.ShapeDtypeStruct((BATCH * MAX_LEN, D_MODEL), jnp.float32),
    mesh=_mesh,
    scratch_types=[
        pltpu.VMEM((STEPS, CHUNK), jnp.int32),
        pltpu.VMEM((NBUF, CHUNK, D_MODEL), jnp.float32),
        pltpu.VMEM((2 * AB_CHUNK,), jnp.float32),
        pltpu.VMEM((2 * G * D_MODEL,), jnp.float32),
        pltpu.SemaphoreType.DMA((NBUF,)),
        pltpu.SemaphoreType.DMA((NBUF,)),
        pltpu.SemaphoreType.DMA((2,)),
    ],
)
def _sc_embed(x_hbm, table_hbm, ab_hbm, cs_hbm, out_hbm,
              idx_all, res_v, ab_v, cs_v, gsem, ssem, absem):
    wid = lax.axis_index("s") * NUM_CORES + lax.axis_index("c")
    s_base = pl.multiple_of(wid * SEQ_PER_WORKER, SEQ_PER_WORKER)
    row_base = pl.multiple_of(wid * CHUNKS_PER_WORKER, CHUNKS_PER_WORKER)

    for b in range(BATCH):
        pltpu.sync_copy(
            x_hbm.at[pl.ds(b * ROWS_PER_BATCH + row_base, CHUNKS_PER_WORKER)],
            idx_all.at[pl.ds(b * CHUNKS_PER_WORKER, CHUNKS_PER_WORKER)])
    pltpu.sync_copy(cs_hbm, cs_v)

    def gather_copy(i, slot):
        b = lax.rem(i, BATCH)
        j = lax.div(i, BATCH)
        return pltpu.make_async_copy(
            table_hbm.at[idx_all.at[b * CHUNKS_PER_WORKER + j]],
            res_v.at[slot], gsem.at[slot])

    def store_copy(i, slot):
        b = lax.rem(i, BATCH)
        j = lax.div(i, BATCH)
        off = pl.multiple_of(b * MAX_LEN + s_base + j * CHUNK, CHUNK)
        return pltpu.make_async_copy(
            res_v.at[slot], out_hbm.at[pl.ds(off, CHUNK)], ssem.at[slot])

    def ab_copy(g, aslot):
        o = pl.multiple_of((wid * (SEQ_PER_WORKER // G) + QPC * g)
                           * 2 * D_MODEL, 2 * D_MODEL)
        return pltpu.make_async_copy(
            ab_hbm.at[pl.ds(o, AB_CHUNK)],
            ab_v.at[pl.ds(aslot * AB_CHUNK, AB_CHUNK)], absem.at[aslot])

    ab_copy(0, 0).start()
    ab_copy(1, 1).start()
    gather_copy(0, 0).start()

    def group(g, carry):
        ga = lax.rem(g, 2)
        ab_copy(g, ga).wait()
        for k in range(BATCH):
            i = g * BATCH + k
            slot = k % NBUF
            nslot = (k + 1) % NBUF

            @pl.when(i + 1 < STEPS)
            def _():
                pl.when(i >= 1)(lambda: store_copy(i - 1, nslot).wait())
                gather_copy(i + 1, nslot).start()

            gather_copy(i, slot).wait()

            for h in range(QPC):
                abase = ga * AB_CHUNK + h * 2 * D_MODEL

                @plsc.parallel_loop(0, D_MODEL // LANES, 1)
                def col_add(kc):
                    a = ab_v[pl.ds(abase + kc * LANES, LANES)]
                    b = ab_v[pl.ds(abase + D_MODEL + kc * LANES, LANES)]
                    for r in range(G):
                        c = cs_v[pl.ds(r * D_MODEL + kc * LANES, LANES)]
                        s = cs_v[pl.ds((G + r) * D_MODEL + kc * LANES, LANES)]
                        plsc.addupdate(
                            res_v.at[slot, h * G + r,
                                     pl.ds(kc * LANES, LANES)],
                            a * c + b * s)

            store_copy(i, slot).start()
        pl.when(g + 2 < CHUNKS_PER_WORKER)(lambda: ab_copy(g + 2, ga).start())
        return carry

    lax.fori_loop(0, CHUNKS_PER_WORKER, group, 0, unroll=False)

    for t in range(STEPS - NBUF, STEPS):
        store_copy(t, t % NBUF).wait()


def kernel(x, table):
    ab, cs = _make_pe_factors()
    idx = x.reshape(BATCH * ROWS_PER_BATCH, CHUNK).astype(jnp.int32)
    out = _sc_embed(idx, table, jnp.asarray(ab), jnp.asarray(cs))
    return out.reshape(BATCH, MAX_LEN, D_MODEL)


# speedup vs baseline: 1.2692x; 1.2692x over previous
import math
import functools

import numpy as np
import jax
import jax.numpy as jnp
from jax import lax
from jax.experimental import pallas as pl
from jax.experimental.pallas import tpu as pltpu
from jax.experimental.pallas import tpu_sc as plsc

D_MODEL = 1024
MAX_LEN = 8192
BATCH = 4
LANES = 16
NUM_CORES = 2
NUM_SUBCORES = 16
NUM_WORKERS = NUM_CORES * NUM_SUBCORES
SEQ_PER_WORKER = MAX_LEN // NUM_WORKERS
CHUNK = 32
G = 16
QPC = CHUNK // G
CHUNKS_PER_WORKER = SEQ_PER_WORKER // CHUNK
STEPS = CHUNKS_PER_WORKER * BATCH
ROWS_PER_BATCH = MAX_LEN // CHUNK
NQ = MAX_LEN // G
NBUF = 2


def _make_pe_factors():
    w = np.exp(np.arange(0, D_MODEL, 2, dtype=np.float64)
               * (-math.log(10000.0) / D_MODEL))
    th = (G * np.arange(NQ, dtype=np.float64))[:, None] * w[None, :]
    ph = np.arange(G, dtype=np.float64)[:, None] * w[None, :]
    a = np.zeros((NQ, D_MODEL), np.float32)
    b = np.zeros((NQ, D_MODEL), np.float32)
    c = np.zeros((G, D_MODEL), np.float32)
    s = np.zeros((G, D_MODEL), np.float32)
    a[:, 0::2], a[:, 1::2] = np.sin(th), np.cos(th)
    b[:, 0::2], b[:, 1::2] = np.cos(th), -np.sin(th)
    c[:, 0::2], c[:, 1::2] = np.cos(ph), np.cos(ph)
    s[:, 0::2], s[:, 1::2] = np.sin(ph), np.sin(ph)
    ab = np.stack([a, b], axis=1).reshape(NQ * 2 * D_MODEL)
    cs = np.concatenate([c, s], axis=0).reshape(2 * G * D_MODEL)
    return ab, cs


_mesh = plsc.VectorSubcoreMesh(
    core_axis_name="c", subcore_axis_name="s",
    num_cores=NUM_CORES, num_subcores=NUM_SUBCORES)

AB_CHUNK = QPC * 2 * D_MODEL


@functools.partial(
    pl.kernel,
    out_type=jax.ShapeDtypeStruct((BATCH * MAX_LEN, D_MODEL), jnp.float32),
    mesh=_mesh,
    scratch_types=[
        pltpu.VMEM((STEPS, CHUNK), jnp.int32),
        pltpu.VMEM((NBUF, CHUNK, D_MODEL), jnp.float32),
        pltpu.VMEM((2 * AB_CHUNK,), jnp.float32),
        pltpu.VMEM((2 * G * D_MODEL,), jnp.float32),
        pltpu.SemaphoreType.DMA((NBUF,)),
        pltpu.SemaphoreType.DMA((NBUF,)),
        pltpu.SemaphoreType.DMA((2,)),
    ],
)
def _sc_embed(x_hbm, table_hbm, ab_hbm, cs_hbm, out_hbm,
              idx_all, res_v, ab_v, cs_v, gsem, ssem, absem):
    wid = lax.axis_index("s") * NUM_CORES + lax.axis_index("c")
    s_base = pl.multiple_of(wid * SEQ_PER_WORKER, SEQ_PER_WORKER)
    row_base = pl.multiple_of(wid * CHUNKS_PER_WORKER, CHUNKS_PER_WORKER)

    for b in range(BATCH):
        pltpu.sync_copy(
            x_hbm.at[pl.ds(b * ROWS_PER_BATCH + row_base, CHUNKS_PER_WORKER)],
            idx_all.at[pl.ds(b * CHUNKS_PER_WORKER, CHUNKS_PER_WORKER)])
    pltpu.sync_copy(cs_hbm, cs_v)

    def gather_copy(i, slot):
        b = lax.rem(i, BATCH)
        j = lax.div(i, BATCH)
        return pltpu.make_async_copy(
            table_hbm.at[idx_all.at[b * CHUNKS_PER_WORKER + j]],
            res_v.at[slot], gsem.at[slot])

    def store_copy(i, slot):
        b = lax.rem(i, BATCH)
        j = lax.div(i, BATCH)
        off = pl.multiple_of(b * MAX_LEN + s_base + j * CHUNK, CHUNK)
        return pltpu.make_async_copy(
            res_v.at[slot], out_hbm.at[pl.ds(off, CHUNK)], ssem.at[slot])

    def ab_copy(g, aslot):
        o = pl.multiple_of((wid * (SEQ_PER_WORKER // G) + QPC * g)
                           * 2 * D_MODEL, 2 * D_MODEL)
        return pltpu.make_async_copy(
            ab_hbm.at[pl.ds(o, AB_CHUNK)],
            ab_v.at[pl.ds(aslot * AB_CHUNK, AB_CHUNK)], absem.at[aslot])

    ab_copy(0, 0).start()
    ab_copy(1, 1).start()
    gather_copy(0, 0).start()

    def group(g, carry):
        ga = lax.rem(g, 2)
        ab_copy(g, ga).wait()
        for k in range(BATCH):
            i = g * BATCH + k
            slot = k % NBUF
            nslot = (k + 1) % NBUF

            @pl.when(i + 1 < STEPS)
            def _():
                pl.when(i >= 1)(lambda: store_copy(i - 1, nslot).wait())
                gather_copy(i + 1, nslot).start()

            gather_copy(i, slot).wait()

            abase = ga * AB_CHUNK

            @plsc.parallel_loop(0, D_MODEL // LANES, 1)
            def col_add(kc):
                sl = pl.ds(kc * LANES, LANES)
                ab = [(ab_v[pl.ds(abase + h * 2 * D_MODEL
                                  + kc * LANES, LANES)],
                       ab_v[pl.ds(abase + h * 2 * D_MODEL + D_MODEL
                                  + kc * LANES, LANES)])
                      for h in range(QPC)]
                for r in range(G):
                    c = cs_v[pl.ds(r * D_MODEL + kc * LANES, LANES)]
                    s = cs_v[pl.ds((G + r) * D_MODEL + kc * LANES, LANES)]
                    for h in range(QPC):
                        plsc.addupdate(res_v.at[slot, h * G + r, sl],
                                       ab[h][0] * c + ab[h][1] * s)

            store_copy(i, slot).start()
        pl.when(g + 2 < CHUNKS_PER_WORKER)(lambda: ab_copy(g + 2, ga).start())
        return carry

    lax.fori_loop(0, CHUNKS_PER_WORKER, group, 0, unroll=False)

    for t in range(STEPS - NBUF, STEPS):
        store_copy(t, t % NBUF).wait()


def kernel(x, table):
    ab, cs = _make_pe_factors()
    idx = x.reshape(BATCH * ROWS_PER_BATCH, CHUNK).astype(jnp.int32)
    out = _sc_embed(idx, table, jnp.asarray(ab), jnp.asarray(cs))
    return out.reshape(BATCH, MAX_LEN, D_MODEL)
